# Initial kernel scaffold; baseline (speedup 1.0000x reference)
import jax
import jax.numpy as jnp
from jax.experimental import pallas as pl


def kernel(x):
    raise NotImplementedError("write your pallas kernel here")


# speedup vs baseline: 1.4179x; 1.4179x over previous
import jax
import jax.numpy as jnp
from jax.experimental import pallas as pl
from jax.experimental.pallas import tpu as pltpu

_KS = (3, 7, 11)
_LAMBDA = 1e-4
_H = 80
_W = 80
_BLK = 8


def _shift(v, d, axis):
    n = v.shape[axis]
    idx = [slice(None)] * v.ndim
    zshape = list(v.shape)
    zshape[axis] = abs(d)
    z = jnp.zeros(zshape, v.dtype)
    if d > 0:
        idx[axis] = slice(0, n - d)
        return jnp.concatenate([z, v[tuple(idx)]], axis=axis)
    else:
        idx[axis] = slice(-d, n)
        return jnp.concatenate([v[tuple(idx)], z], axis=axis)


def _box_w(v):
    b3 = v + _shift(v, 1, 2) + _shift(v, -1, 2)
    b7 = b3 + _shift(v, 2, 2) + _shift(v, -2, 2) + _shift(v, 3, 2) + _shift(v, -3, 2)
    b11 = b7 + _shift(v, 4, 2) + _shift(v, -4, 2) + _shift(v, 5, 2) + _shift(v, -5, 2)
    return b3, b7, b11


def _box_h(v, k):
    p = k // 2
    acc = v
    for d in range(1, p + 1):
        acc = acc + _shift(v, d, 1) + _shift(v, -d, 1)
    return acc


def _camsam_body(x_ref, o_ref):
    x = x_ref[...]
    x2 = x * x
    xw3, xw7, xw11 = _box_w(x)
    sw3, sw7, sw11 = _box_w(x2)
    energy = None
    for k, xw, sw in ((3, xw3, sw3), (7, xw7, sw7), (11, xw11, sw11)):
        inv = 1.0 / float(k * k)
        m = _box_h(xw, k) * inv
        m2 = _box_h(sw, k) * inv
        var = m2 - m * m
        num = (x - m) * (x - m)
        den = 4.0 * (var + _LAMBDA)
        e = num / den
        energy = e if energy is None else energy + e
    fused = energy * (1.0 / 3.0) + 0.5
    att = jax.nn.sigmoid(1.0 - fused)
    o_ref[...] = x * att


def kernel(x):
    n, c, h, w = x.shape
    planes = n * c
    xf = x.reshape(planes, h, w)
    grid = (planes // _BLK,)
    out = pl.pallas_call(
        _camsam_body,
        grid=grid,
        in_specs=[pl.BlockSpec((_BLK, h, w), lambda i: (i, 0, 0))],
        out_specs=pl.BlockSpec((_BLK, h, w), lambda i: (i, 0, 0)),
        out_shape=jax.ShapeDtypeStruct((planes, h, w), x.dtype),
        compiler_params=pltpu.CompilerParams(
            dimension_semantics=("parallel",),
        ),
    )(xf)
    return out.reshape(n, c, h, w)


# speedup vs baseline: 5.9970x; 4.2294x over previous
import jax
import jax.numpy as jnp
from jax.experimental import pallas as pl
from jax.experimental.pallas import tpu as pltpu

_LAMBDA = 1e-4
_CBLK = 128


def _shift_w(v, d):
    n = v.shape[1]
    zshape = (v.shape[0], abs(d), v.shape[2])
    z = jnp.zeros(zshape, v.dtype)
    if d > 0:
        return jnp.concatenate([z, v[:, : n - d, :]], axis=1)
    return jnp.concatenate([v[:, -d:, :], z], axis=1)


def _box_h(v, p):
    nh = v.shape[0]
    acc = v[0:1]
    cums = [acc]
    for h in range(1, nh):
        acc = acc + v[h : h + 1]
        cums.append(acc)
    out = []
    for h in range(nh):
        hi = cums[min(h + p, nh - 1)]
        lo_idx = h - p - 1
        out.append(hi - cums[lo_idx] if lo_idx >= 0 else hi)
    return jnp.concatenate(out, axis=0)


def _camsam_body(x_ref, o_ref):
    x = x_ref[...]
    x2 = x * x
    xacc = x + _shift_w(x, 1) + _shift_w(x, -1)
    sacc = x2 + _shift_w(x2, 1) + _shift_w(x2, -1)
    energy = None
    for k in (3, 7, 11):
        if k > 3:
            for d in (k // 2 - 1, k // 2):
                xacc = xacc + _shift_w(x, d) + _shift_w(x, -d)
                sacc = sacc + _shift_w(x2, d) + _shift_w(x2, -d)
        p = k // 2
        inv = 1.0 / float(k * k)
        m = _box_h(xacc, p) * inv
        m2 = _box_h(sacc, p) * inv
        var = m2 - m * m
        num = (x - m) * (x - m)
        e = num / (4.0 * (var + _LAMBDA))
        energy = e if energy is None else energy + e
    att = jax.nn.sigmoid(0.5 - energy * (1.0 / 3.0))
    o_ref[...] = x * att


def kernel(x):
    n, c, h, w = x.shape
    xt = jnp.transpose(x, (0, 2, 3, 1)).reshape(n * h, w, c)
    csplit = c // _CBLK
    grid = (n * csplit,)
    out = pl.pallas_call(
        _camsam_body,
        grid=grid,
        in_specs=[
            pl.BlockSpec((h, w, _CBLK), lambda i, s=csplit: (i // s, 0, i % s))
        ],
        out_specs=pl.BlockSpec(
            (h, w, _CBLK), lambda i, s=csplit: (i // s, 0, i % s)
        ),
        out_shape=jax.ShapeDtypeStruct((n * h, w, c), x.dtype),
        compiler_params=pltpu.CompilerParams(
            dimension_semantics=("parallel",),
            vmem_limit_bytes=100 * 1024 * 1024,
        ),
    )(xt)
    return jnp.transpose(out.reshape(n, h, w, c), (0, 3, 1, 2))


# speedup vs baseline: 6.6495x; 1.1088x over previous
import jax
import jax.numpy as jnp
import numpy as np
from jax.experimental import pallas as pl
from jax.experimental.pallas import tpu as pltpu
from jax.sharding import Mesh, PartitionSpec as P

_LAMBDA = 1e-4
_CBLK = 128


def _shift_w(v, d):
    n = v.shape[1]
    zshape = (v.shape[0], abs(d), v.shape[2])
    z = jnp.zeros(zshape, v.dtype)
    if d > 0:
        return jnp.concatenate([z, v[:, : n - d, :]], axis=1)
    return jnp.concatenate([v[:, -d:, :], z], axis=1)


def _box_w3(v):
    vl = _shift_w(v, 1)
    vr = _shift_w(v, -1)
    s1l = v + vl
    s1r = v + vr
    b3 = s1l + vr
    b7 = b3 + _shift_w(s1l, 2) + _shift_w(s1r, -2)
    b11 = b7 + _shift_w(s1l, 4) + _shift_w(s1r, -4)
    return b3, b7, b11


def _box_h(v, p):
    nh = v.shape[0]
    acc = v[0:1]
    cums = [acc]
    for h in range(1, nh):
        acc = acc + v[h : h + 1]
        cums.append(acc)
    out = []
    for h in range(nh):
        hi = cums[min(h + p, nh - 1)]
        lo_idx = h - p - 1
        out.append(hi - cums[lo_idx] if lo_idx >= 0 else hi)
    return jnp.concatenate(out, axis=0)


def _camsam_body(x_ref, o_ref):
    x = x_ref[...]
    x2 = x * x
    xw = _box_w3(x)
    sw = _box_w3(x2)
    energy = None
    for i, k in enumerate((3, 7, 11)):
        p = k // 2
        inv = 1.0 / float(k * k)
        m = _box_h(xw[i], p) * inv
        m2 = _box_h(sw[i], p) * inv
        var = m2 - m * m
        num = (x - m) * (x - m)
        e = num / (4.0 * (var + _LAMBDA))
        energy = e if energy is None else energy + e
    att = jax.nn.sigmoid(0.5 - energy * (1.0 / 3.0))
    o_ref[...] = x * att


def _forward(x):
    n, c, h, w = x.shape
    xt = jnp.transpose(x, (0, 2, 3, 1)).reshape(n * h, w, c)
    csplit = c // _CBLK
    grid = (n * csplit,)
    out = pl.pallas_call(
        _camsam_body,
        grid=grid,
        in_specs=[
            pl.BlockSpec((h, w, _CBLK), lambda i, s=csplit: (i // s, 0, i % s))
        ],
        out_specs=pl.BlockSpec(
            (h, w, _CBLK), lambda i, s=csplit: (i // s, 0, i % s)
        ),
        out_shape=jax.ShapeDtypeStruct((n * h, w, c), x.dtype),
        compiler_params=pltpu.CompilerParams(
            dimension_semantics=("parallel",),
            vmem_limit_bytes=100 * 1024 * 1024,
        ),
    )(xt)
    return jnp.transpose(out.reshape(n, h, w, c), (0, 3, 1, 2))


def kernel(x):
    return _forward(x)
